# baseline (speedup 1.0000x reference)
import functools

import jax
import jax.numpy as jnp
from jax import lax
from jax.experimental import pallas as pl
from jax.experimental.pallas import tpu as pltpu
from jax.experimental.pallas import tpu_sc as plsc

N_NODES = 50000
N_EDGES = 800000
D = 64
R = 8
DH = D // 2

NC = 2
NS = 16
L = 16

EB = 128
EPT = 392 * EB
EPAD = NS * EPT
NB = EPT // EB

AGG_ROWS = 50176
PER_S = AGG_ROWS // NS
CNT_ROWS = 401408
CNT_PER_S = CNT_ROWS // NS
EPW = EPAD // (NC * NS)
NBW = EPW // EB

_ROW_TILE = 1000

_mesh = plsc.VectorSubcoreMesh(core_axis_name="c", subcore_axis_name="s")

_BCAST_DNUMS = lax.GatherDimensionNumbers(
    offset_dims=(), collapsed_slice_dims=(0,), start_index_map=(0,))


def _lane_bcast(v, j):
    idx = jnp.full((L, 1), j, jnp.int32)
    return lax.gather(v, idx, _BCAST_DNUMS, (1,),
                      mode=lax.GatherScatterMode.PROMISE_IN_BOUNDS)


def _sc_count(comb, val, zblk):
    @functools.partial(
        pl.kernel,
        out_type=jax.ShapeDtypeStruct((CNT_ROWS,), jnp.float32),
        mesh=_mesh,
        compiler_params=pltpu.CompilerParams(use_tc_tiling_on_sc=False),
        scratch_types=[
            pltpu.VMEM_SHARED((CNT_ROWS,), jnp.float32),
            pltpu.VMEM((EB,), jnp.int32),
            pltpu.VMEM((EB,), jnp.float32),
        ],
    )
    def k(comb_h, val_h, z_h, out_h, cnt_sh, comb_v, val_v):
        c = lax.axis_index("c")
        s = lax.axis_index("s")

        @pl.when(c == 0)
        def _():
            pltpu.sync_copy(z_h, cnt_sh.at[pl.ds(s * CNT_PER_S, CNT_PER_S)])
            plsc.subcore_barrier()

            def body(b, _):
                e0 = s * EPT + b * EB
                pltpu.sync_copy(comb_h.at[pl.ds(e0, EB)], comb_v)
                pltpu.sync_copy(val_h.at[pl.ds(e0, EB)], val_v)
                pltpu.sync_copy(val_v, cnt_sh.at[comb_v], add=True)
                return ()

            lax.fori_loop(0, NB, body, ())
            plsc.subcore_barrier()
            pltpu.sync_copy(
                cnt_sh.at[pl.ds(s * CNT_PER_S, CNT_PER_S)],
                out_h.at[pl.ds(s * CNT_PER_S, CNT_PER_S)],
            )

    return k(comb, val, zblk)


def _sc_inv_gather(comb, invp):
    @functools.partial(
        pl.kernel,
        out_type=jax.ShapeDtypeStruct((EPAD,), jnp.float32),
        mesh=_mesh,
        compiler_params=pltpu.CompilerParams(use_tc_tiling_on_sc=False),
        scratch_types=[
            pltpu.VMEM((EB,), jnp.int32),
            pltpu.VMEM((EB,), jnp.float32),
        ],
    )
    def k(comb_h, invp_h, out_h, comb_v, inv_v):
        c = lax.axis_index("c")
        s = lax.axis_index("s")
        base = (s * NC + c) * EPW

        def body(b, _):
            e0 = base + b * EB
            pltpu.sync_copy(comb_h.at[pl.ds(e0, EB)], comb_v)
            pltpu.sync_copy(invp_h.at[comb_v], inv_v)
            pltpu.sync_copy(inv_v, out_h.at[pl.ds(e0, EB)])
            return ()

        lax.fori_loop(0, NBW, body, ())

    return k(comb, invp)


def _sc_conv(tbl, epk, invE, zblk):
    @functools.partial(
        pl.kernel,
        out_type=jax.ShapeDtypeStruct((NC, AGG_ROWS, DH), jnp.float32),
        mesh=_mesh,
        compiler_params=pltpu.CompilerParams(use_tc_tiling_on_sc=False),
        scratch_types=[
            pltpu.VMEM_SHARED((AGG_ROWS, DH), jnp.float32),
            pltpu.VMEM((4, 2 * EB), jnp.int32),
            pltpu.VMEM((4, EB), jnp.float32),
            pltpu.VMEM((4, EB), jnp.int32),
            pltpu.VMEM((4, EB), jnp.int32),
            pltpu.VMEM((2, EB, DH), jnp.float32),
            pltpu.VMEM((2, EB, DH), jnp.float32),
            pltpu.SemaphoreType.DMA((4,)),
            pltpu.SemaphoreType.DMA((4,)),
            pltpu.SemaphoreType.DMA((4,)),
        ],
    )
    def k(tbl_h, epk_h, inv_h, z_h, out_h,
          agg_sh, in_v, inf_v, idx_v, sidx_v, rows_v, srow_v,
          insem, gsem, ssem):
        c = lax.axis_index("c")
        s = lax.axis_index("s")

        pltpu.sync_copy(z_h, agg_sh.at[pl.ds(s * PER_S, PER_S)])
        plsc.subcore_barrier()

        def issue_in(b, j):
            e0 = (s * NB + b) * (2 * EB)
            pltpu.async_copy(epk_h.at[pl.ds(e0, 2 * EB)], in_v.at[j],
                             insem.at[j])
            f0 = (s * NB + b) * EB
            pltpu.async_copy(inv_h.at[pl.ds(f0, EB)], inf_v.at[j],
                             insem.at[j])

        def wait_in(b, j):
            e0 = (s * NB + b) * (2 * EB)
            pltpu.make_async_copy(epk_h.at[pl.ds(e0, 2 * EB)], in_v.at[j],
                                  insem.at[j]).wait()
            f0 = (s * NB + b) * EB
            pltpu.make_async_copy(inv_h.at[pl.ds(f0, EB)], inf_v.at[j],
                                  insem.at[j]).wait()

        def prep(j):
            for kk in range(EB // L):
                sl = pl.ds(kk * L, L)
                idx_v[j, sl] = in_v[j, sl] + c
                sidx_v[j, sl] = in_v[j, pl.ds(EB + kk * L, L)]

        def issue_gather(j):
            pltpu.async_copy(tbl_h.at[idx_v.at[j]], rows_v.at[j % 2],
                             gsem.at[j])

        def wait_gather(j):
            pltpu.make_async_copy(tbl_h.at[idx_v.at[j]], rows_v.at[j % 2],
                                  gsem.at[j]).wait()

        def scale(j):
            j2 = j % 2
            for kk in range(EB // L):
                iv = inf_v[j, pl.ds(kk * L, L)]
                for jj in range(L):
                    e = kk * L + jj
                    bc = _lane_bcast(iv, jj)
                    srow_v[j2, e, pl.ds(0, L)] = \
                        rows_v[j2, e, pl.ds(0, L)] * bc
                    srow_v[j2, e, pl.ds(L, L)] = \
                        rows_v[j2, e, pl.ds(L, L)] * bc

        def issue_scatter(j):
            pltpu.async_copy(srow_v.at[j % 2], agg_sh.at[sidx_v.at[j]],
                             ssem.at[j], add=True)

        def wait_scatter(j):
            pltpu.make_async_copy(srow_v.at[j % 2], agg_sh.at[sidx_v.at[j]],
                                  ssem.at[j]).wait()

        issue_in(0, 0)
        issue_in(1, 1)
        issue_in(2, 2)
        wait_in(0, 0)
        prep(0)
        issue_gather(0)

        def body(g, _):
            for j in range(4):
                b = 4 * g + j
                jn = (j + 1) % 4
                wait_in(b + 1, jn)
                prep(jn)
                issue_gather(jn)
                wait_gather(j)

                @pl.when(b >= 2)
                def _():
                    wait_scatter((j + 2) % 4)

                scale(j)
                issue_scatter(j)

                @pl.when(b < NB - 3)
                def _():
                    issue_in(b + 3, (j + 3) % 4)
            return ()

        lax.fori_loop(0, NB // 4 - 1, body, ())
        issue_in(NB - 1, (NB - 1) % 4)
        for j in range(4):
            if j < 3:
                wait_in(NB - 3 + j, (j + 1) % 4)
                prep((j + 1) % 4)
                issue_gather((j + 1) % 4)
            wait_gather(j)
            wait_scatter((j + 2) % 4)
            scale(j)
            issue_scatter(j)
        wait_scatter(2)
        wait_scatter(3)

        plsc.subcore_barrier()
        pltpu.sync_copy(
            agg_sh.at[pl.ds(s * PER_S, PER_S)],
            out_h.at[c, pl.ds(s * PER_S, PER_S)],
        )

    return k(tbl, epk, invE, zblk)


def _sum_parts(parts, refs):
    h = None
    for (tag, _), r in zip(parts, refs):
        t = (jnp.concatenate([r[0], r[1]], axis=-1) if tag == 'agg'
             else r[...])
        h = t if h is None else h + t
    return h


def _part_spec(tag):
    if tag == 'agg':
        return pl.BlockSpec((NC, _ROW_TILE, DH), lambda i: (0, i, 0))
    return pl.BlockSpec((_ROW_TILE, D), lambda i: (i, 0))


def _dense_stage(parts, lng, lnb, Wst, root, bias, want_hmat):
    n_parts = len(parts)

    def body(*refs):
        pr = refs[:n_parts]
        lng_r, lnb_r, w_r, root_r, bias_r = refs[n_parts:n_parts + 5]
        o_t = refs[n_parts + 5]
        o_rp = refs[n_parts + 6]
        h = _sum_parts(parts, pr)
        if want_hmat:
            refs[n_parts + 7][...] = h
        m = jnp.mean(h, axis=-1, keepdims=True)
        v = jnp.mean((h - m) * (h - m), axis=-1, keepdims=True)
        xn = (h - m) * jax.lax.rsqrt(v + 1e-5) * lng_r[...] + lnb_r[...]
        y = xn * 0.5 * (1.0 + jax.lax.erf(xn / jnp.sqrt(2.0).astype(xn.dtype)))
        for ri in range(R):
            o_t[ri] = jnp.dot(y, w_r[ri], precision=lax.Precision.HIGHEST,
                              preferred_element_type=jnp.float32)
        o_rp[...] = jnp.dot(y, root_r[...], precision=lax.Precision.HIGHEST,
                            preferred_element_type=jnp.float32) + bias_r[...]

    in_specs = [_part_spec(tag) for tag, _ in parts] + [
        pl.BlockSpec((1, D), lambda i: (0, 0)),
        pl.BlockSpec((1, D), lambda i: (0, 0)),
        pl.BlockSpec((R, D, D), lambda i: (0, 0, 0)),
        pl.BlockSpec((D, D), lambda i: (0, 0)),
        pl.BlockSpec((1, D), lambda i: (0, 0)),
    ]
    out_shape = [jax.ShapeDtypeStruct((R, N_NODES, D), jnp.float32),
                 jax.ShapeDtypeStruct((N_NODES, D), jnp.float32)]
    out_specs = [pl.BlockSpec((R, _ROW_TILE, D), lambda i: (0, i, 0)),
                 pl.BlockSpec((_ROW_TILE, D), lambda i: (i, 0))]
    if want_hmat:
        out_shape.append(jax.ShapeDtypeStruct((N_NODES, D), jnp.float32))
        out_specs.append(pl.BlockSpec((_ROW_TILE, D), lambda i: (i, 0)))
    return pl.pallas_call(
        body,
        grid=(N_NODES // _ROW_TILE,),
        in_specs=in_specs,
        out_specs=out_specs,
        out_shape=out_shape,
    )(*[a for _, a in parts], lng.reshape(1, D), lnb.reshape(1, D),
      Wst, root, bias.reshape(1, D))


def _final_stage(parts):
    n_parts = len(parts)

    def body(*refs):
        o_cs = refs[n_parts]
        h = _sum_parts(parts, refs[:n_parts])

        @pl.when(pl.program_id(0) == 0)
        def _():
            o_cs[...] = jnp.zeros((1, D), jnp.float32)

        o_cs[...] += jnp.sum(h, axis=0, keepdims=True)

    return pl.pallas_call(
        body,
        grid=(N_NODES // _ROW_TILE,),
        in_specs=[_part_spec(tag) for tag, _ in parts],
        out_specs=pl.BlockSpec((1, D), lambda i: (0, 0)),
        out_shape=jax.ShapeDtypeStruct((1, D), jnp.float32),
    )(*[a for _, a in parts])


def kernel(x, edge_index, edge_attr, params):
    src = edge_index[0].astype(jnp.int32)
    dst = edge_index[1].astype(jnp.int32)
    et = edge_attr.astype(jnp.int32)

    pad = EPAD - N_EDGES
    comb = jnp.concatenate([dst * R + et, jnp.full((pad,), N_NODES * R, jnp.int32)])
    g2 = jnp.concatenate([(et * N_NODES + src) * NC, jnp.zeros((pad,), jnp.int32)])
    dstp = jnp.concatenate([dst, jnp.full((pad,), N_NODES, jnp.int32)])
    val = jnp.concatenate([jnp.ones((N_EDGES,), jnp.float32),
                           jnp.zeros((pad,), jnp.float32)])

    zcnt = jnp.zeros((CNT_PER_S,), jnp.float32)
    zagg = jnp.zeros((PER_S, DH), jnp.float32)

    cnt = _sc_count(comb, val, zcnt)
    invp = 1.0 / jnp.maximum(cnt, 1.0)
    invE = _sc_inv_gather(comb, invp)
    epk = jnp.stack([g2.reshape(-1, EB), dstp.reshape(-1, EB)],
                    axis=1).reshape(-1)

    emb = params['emb']
    h = jnp.where((x[:, None] == 1), emb[1][None, :], emb[0][None, :])

    for bi, p in enumerate(params['blocks']):
        if bi == 0:
            parts1 = [('full', h)]
        else:
            parts1 = [('agg', agg2), ('full', rp2), ('full', h)]
        o1 = _dense_stage(parts1, p['ln1_g'], p['ln1_b'], p['W1'],
                          p['root1'], p['b1'], want_hmat=(bi > 0))
        if bi > 0:
            t1, rp1, h = o1
        else:
            t1, rp1 = o1
        agg1 = _sc_conv(t1.reshape(R * N_NODES * NC, DH), epk, invE, zagg)
        t2, rp2 = _dense_stage([('agg', agg1), ('full', rp1)],
                               p['ln2_g'], p['ln2_b'], p['W2'],
                               p['root2'], p['b2'], want_hmat=False)
        agg2 = _sc_conv(t2.reshape(R * N_NODES * NC, DH), epk, invE, zagg)

    cs = _final_stage([('agg', agg2), ('full', rp2), ('full', h)])
    pooled = (cs[0] / N_NODES) @ params['cr_W'] + params['cr_b']
    z = jax.nn.gelu(pooled @ params['p1_W'] + params['p1_b'], approximate=False)
    return z @ params['p2_W'] + params['p2_b']


# speedup vs baseline: 1.4498x; 1.4498x over previous
import functools

import jax
import jax.numpy as jnp
from jax import lax
from jax.experimental import pallas as pl
from jax.experimental.pallas import tpu as pltpu
from jax.experimental.pallas import tpu_sc as plsc

N_NODES = 50000
N_EDGES = 800000
D = 64
R = 8
DH = D // 2

NC = 2
NS = 16
L = 16

EB = 128
EPT = 392 * EB
EPAD = NS * EPT
NB = EPT // EB

AGG_ROWS = 50176
PER_S = AGG_ROWS // NS
CNT_ROWS = 401408
CNT_PER_S = CNT_ROWS // NS
EPW = EPAD // (NC * NS)
NBW = EPW // EB

_ROW_TILE = 1000

_mesh = plsc.VectorSubcoreMesh(core_axis_name="c", subcore_axis_name="s")

_BCAST_DNUMS = lax.GatherDimensionNumbers(
    offset_dims=(), collapsed_slice_dims=(0,), start_index_map=(0,))


def _lane_bcast(v, j):
    idx = jnp.full((L, 1), j, jnp.int32)
    return lax.gather(v, idx, _BCAST_DNUMS, (1,),
                      mode=lax.GatherScatterMode.PROMISE_IN_BOUNDS)


def _sc_count(comb, val, zblk):
    @functools.partial(
        pl.kernel,
        out_type=jax.ShapeDtypeStruct((CNT_ROWS,), jnp.float32),
        mesh=_mesh,
        compiler_params=pltpu.CompilerParams(use_tc_tiling_on_sc=False),
        scratch_types=[
            pltpu.VMEM_SHARED((CNT_ROWS,), jnp.float32),
            pltpu.VMEM((EB,), jnp.int32),
            pltpu.VMEM((EB,), jnp.float32),
        ],
    )
    def k(comb_h, val_h, z_h, out_h, cnt_sh, comb_v, val_v):
        c = lax.axis_index("c")
        s = lax.axis_index("s")

        @pl.when(c == 0)
        def _():
            pltpu.sync_copy(z_h, cnt_sh.at[pl.ds(s * CNT_PER_S, CNT_PER_S)])
            plsc.subcore_barrier()

            def body(b, _):
                e0 = s * EPT + b * EB
                pltpu.sync_copy(comb_h.at[pl.ds(e0, EB)], comb_v)
                pltpu.sync_copy(val_h.at[pl.ds(e0, EB)], val_v)
                pltpu.sync_copy(val_v, cnt_sh.at[comb_v], add=True)
                return ()

            lax.fori_loop(0, NB, body, ())
            plsc.subcore_barrier()
            pltpu.sync_copy(
                cnt_sh.at[pl.ds(s * CNT_PER_S, CNT_PER_S)],
                out_h.at[pl.ds(s * CNT_PER_S, CNT_PER_S)],
            )

    return k(comb, val, zblk)


def _sc_inv_gather(comb, invp):
    @functools.partial(
        pl.kernel,
        out_type=jax.ShapeDtypeStruct((EPAD,), jnp.float32),
        mesh=_mesh,
        compiler_params=pltpu.CompilerParams(use_tc_tiling_on_sc=False),
        scratch_types=[
            pltpu.VMEM((EB,), jnp.int32),
            pltpu.VMEM((EB,), jnp.float32),
        ],
    )
    def k(comb_h, invp_h, out_h, comb_v, inv_v):
        c = lax.axis_index("c")
        s = lax.axis_index("s")
        base = (s * NC + c) * EPW

        def body(b, _):
            e0 = base + b * EB
            pltpu.sync_copy(comb_h.at[pl.ds(e0, EB)], comb_v)
            pltpu.sync_copy(invp_h.at[comb_v], inv_v)
            pltpu.sync_copy(inv_v, out_h.at[pl.ds(e0, EB)])
            return ()

        lax.fori_loop(0, NBW, body, ())

    return k(comb, invp)


def _sc_conv(tbl, epk, invE, zblk):
    @functools.partial(
        pl.kernel,
        out_type=jax.ShapeDtypeStruct((NC, AGG_ROWS, DH), jnp.float32),
        mesh=_mesh,
        compiler_params=pltpu.CompilerParams(use_tc_tiling_on_sc=False),
        scratch_types=[
            pltpu.VMEM_SHARED((AGG_ROWS, DH), jnp.float32),
            pltpu.VMEM((4, 2 * EB), jnp.int32),
            pltpu.VMEM((4, EB), jnp.float32),
            pltpu.VMEM((4, EB), jnp.int32),
            pltpu.VMEM((4, EB), jnp.int32),
            pltpu.VMEM((2, EB, DH), jnp.float32),
            pltpu.VMEM((2, EB, DH), jnp.float32),
            pltpu.SemaphoreType.DMA((4,)),
            pltpu.SemaphoreType.DMA((4,)),
            pltpu.SemaphoreType.DMA((4,)),
        ],
    )
    def k(tbl_h, epk_h, inv_h, z_h, out_h,
          agg_sh, in_v, inf_v, idx_v, sidx_v, rows_v, srow_v,
          insem, gsem, ssem):
        c = lax.axis_index("c")
        s = lax.axis_index("s")

        pltpu.sync_copy(z_h, agg_sh.at[pl.ds(s * PER_S, PER_S)])
        plsc.subcore_barrier()

        def issue_in(b, j):
            e0 = (s * NB + b) * (2 * EB)
            pltpu.async_copy(epk_h.at[pl.ds(e0, 2 * EB)], in_v.at[j],
                             insem.at[j])
            f0 = (s * NB + b) * EB
            pltpu.async_copy(inv_h.at[pl.ds(f0, EB)], inf_v.at[j],
                             insem.at[j])

        def wait_in(b, j):
            e0 = (s * NB + b) * (2 * EB)
            pltpu.make_async_copy(epk_h.at[pl.ds(e0, 2 * EB)], in_v.at[j],
                                  insem.at[j]).wait()
            f0 = (s * NB + b) * EB
            pltpu.make_async_copy(inv_h.at[pl.ds(f0, EB)], inf_v.at[j],
                                  insem.at[j]).wait()

        def prep(j):
            for kk in range(EB // L):
                sl = pl.ds(kk * L, L)
                idx_v[j, sl] = in_v[j, sl] + c
                sidx_v[j, sl] = in_v[j, pl.ds(EB + kk * L, L)]

        def issue_gather(j):
            pltpu.async_copy(tbl_h.at[idx_v.at[j]], rows_v.at[j % 2],
                             gsem.at[j])

        def wait_gather(j):
            pltpu.make_async_copy(tbl_h.at[idx_v.at[j]], rows_v.at[j % 2],
                                  gsem.at[j]).wait()

        def scale(j):
            j2 = j % 2
            for kk in range(EB // L):
                iv = inf_v[j, pl.ds(kk * L, L)]
                for jj in range(L):
                    e = kk * L + jj
                    bc = _lane_bcast(iv, jj)
                    srow_v[j2, e, pl.ds(0, L)] = \
                        rows_v[j2, e, pl.ds(0, L)] * bc
                    srow_v[j2, e, pl.ds(L, L)] = \
                        rows_v[j2, e, pl.ds(L, L)] * bc

        def issue_scatter(j):
            pltpu.async_copy(srow_v.at[j % 2], agg_sh.at[sidx_v.at[j]],
                             ssem.at[j], add=True)

        def wait_scatter(j):
            pltpu.make_async_copy(srow_v.at[j % 2], agg_sh.at[sidx_v.at[j]],
                                  ssem.at[j]).wait()

        issue_in(0, 0)
        issue_in(1, 1)
        issue_in(2, 2)
        wait_in(0, 0)
        prep(0)
        issue_gather(0)

        def body(g, _):
            for j in range(4):
                b = 4 * g + j
                jn = (j + 1) % 4
                wait_in(b + 1, jn)
                prep(jn)
                issue_gather(jn)
                wait_gather(j)

                @pl.when(b >= 2)
                def _():
                    wait_scatter((j + 2) % 4)

                scale(j)
                issue_scatter(j)

                @pl.when(b < NB - 3)
                def _():
                    issue_in(b + 3, (j + 3) % 4)
            return ()

        lax.fori_loop(0, NB // 4 - 1, body, ())
        issue_in(NB - 1, (NB - 1) % 4)
        for j in range(4):
            if j < 3:
                wait_in(NB - 3 + j, (j + 1) % 4)
                prep((j + 1) % 4)
                issue_gather((j + 1) % 4)
            wait_gather(j)
            wait_scatter((j + 2) % 4)
            scale(j)
            issue_scatter(j)
        wait_scatter(2)
        wait_scatter(3)

        plsc.subcore_barrier()
        pltpu.sync_copy(
            agg_sh.at[pl.ds(s * PER_S, PER_S)],
            out_h.at[c, pl.ds(s * PER_S, PER_S)],
        )

    return k(tbl, epk, invE, zblk)


def _sum_parts(parts, refs):
    h = None
    for (tag, _), r in zip(parts, refs):
        t = (jnp.concatenate([r[0], r[1]], axis=-1) if tag == 'agg'
             else r[...])
        h = t if h is None else h + t
    return h


def _part_spec(tag):
    if tag == 'agg':
        return pl.BlockSpec((NC, _ROW_TILE, DH), lambda i: (0, i, 0))
    return pl.BlockSpec((_ROW_TILE, D), lambda i: (i, 0))


def _dense_stage(parts, lng, lnb, Wst, root, bias, want_hmat):
    n_parts = len(parts)

    def body(*refs):
        pr = refs[:n_parts]
        lng_r, lnb_r, w_r, root_r, bias_r = refs[n_parts:n_parts + 5]
        o_t = refs[n_parts + 5]
        o_rp = refs[n_parts + 6]
        h = _sum_parts(parts, pr)
        if want_hmat:
            refs[n_parts + 7][...] = h
        m = jnp.mean(h, axis=-1, keepdims=True)
        v = jnp.mean((h - m) * (h - m), axis=-1, keepdims=True)
        xn = (h - m) * jax.lax.rsqrt(v + 1e-5) * lng_r[...] + lnb_r[...]
        y = xn * 0.5 * (1.0 + jax.lax.erf(xn / jnp.sqrt(2.0).astype(xn.dtype)))
        for ri in range(R):
            o_t[ri] = jnp.dot(y, w_r[ri], preferred_element_type=jnp.float32)
        o_rp[...] = jnp.dot(y, root_r[...],
                            preferred_element_type=jnp.float32) + bias_r[...]

    in_specs = [_part_spec(tag) for tag, _ in parts] + [
        pl.BlockSpec((1, D), lambda i: (0, 0)),
        pl.BlockSpec((1, D), lambda i: (0, 0)),
        pl.BlockSpec((R, D, D), lambda i: (0, 0, 0)),
        pl.BlockSpec((D, D), lambda i: (0, 0)),
        pl.BlockSpec((1, D), lambda i: (0, 0)),
    ]
    out_shape = [jax.ShapeDtypeStruct((R, N_NODES, D), jnp.float32),
                 jax.ShapeDtypeStruct((N_NODES, D), jnp.float32)]
    out_specs = [pl.BlockSpec((R, _ROW_TILE, D), lambda i: (0, i, 0)),
                 pl.BlockSpec((_ROW_TILE, D), lambda i: (i, 0))]
    if want_hmat:
        out_shape.append(jax.ShapeDtypeStruct((N_NODES, D), jnp.float32))
        out_specs.append(pl.BlockSpec((_ROW_TILE, D), lambda i: (i, 0)))
    return pl.pallas_call(
        body,
        grid=(N_NODES // _ROW_TILE,),
        in_specs=in_specs,
        out_specs=out_specs,
        out_shape=out_shape,
    )(*[a for _, a in parts], lng.reshape(1, D), lnb.reshape(1, D),
      Wst, root, bias.reshape(1, D))


def _final_stage(parts):
    n_parts = len(parts)

    def body(*refs):
        o_cs = refs[n_parts]
        h = _sum_parts(parts, refs[:n_parts])

        @pl.when(pl.program_id(0) == 0)
        def _():
            o_cs[...] = jnp.zeros((1, D), jnp.float32)

        o_cs[...] += jnp.sum(h, axis=0, keepdims=True)

    return pl.pallas_call(
        body,
        grid=(N_NODES // _ROW_TILE,),
        in_specs=[_part_spec(tag) for tag, _ in parts],
        out_specs=pl.BlockSpec((1, D), lambda i: (0, 0)),
        out_shape=jax.ShapeDtypeStruct((1, D), jnp.float32),
    )(*[a for _, a in parts])


def kernel(x, edge_index, edge_attr, params):
    src = edge_index[0].astype(jnp.int32)
    dst = edge_index[1].astype(jnp.int32)
    et = edge_attr.astype(jnp.int32)

    pad = EPAD - N_EDGES
    comb = jnp.concatenate([dst * R + et, jnp.full((pad,), N_NODES * R, jnp.int32)])
    g2 = jnp.concatenate([(et * N_NODES + src) * NC, jnp.zeros((pad,), jnp.int32)])
    dstp = jnp.concatenate([dst, jnp.full((pad,), N_NODES, jnp.int32)])
    val = jnp.concatenate([jnp.ones((N_EDGES,), jnp.float32),
                           jnp.zeros((pad,), jnp.float32)])

    zcnt = jnp.zeros((CNT_PER_S,), jnp.float32)
    zagg = jnp.zeros((PER_S, DH), jnp.float32)

    cnt = _sc_count(comb, val, zcnt)
    invp = 1.0 / jnp.maximum(cnt, 1.0)
    invE = _sc_inv_gather(comb, invp)
    epk = jnp.stack([g2.reshape(-1, EB), dstp.reshape(-1, EB)],
                    axis=1).reshape(-1)

    emb = params['emb']
    h = jnp.where((x[:, None] == 1), emb[1][None, :], emb[0][None, :])

    for bi, p in enumerate(params['blocks']):
        if bi == 0:
            parts1 = [('full', h)]
        else:
            parts1 = [('agg', agg2), ('full', rp2), ('full', h)]
        o1 = _dense_stage(parts1, p['ln1_g'], p['ln1_b'], p['W1'],
                          p['root1'], p['b1'], want_hmat=(bi > 0))
        if bi > 0:
            t1, rp1, h = o1
        else:
            t1, rp1 = o1
        agg1 = _sc_conv(t1.reshape(R * N_NODES * NC, DH), epk, invE, zagg)
        t2, rp2 = _dense_stage([('agg', agg1), ('full', rp1)],
                               p['ln2_g'], p['ln2_b'], p['W2'],
                               p['root2'], p['b2'], want_hmat=False)
        agg2 = _sc_conv(t2.reshape(R * N_NODES * NC, DH), epk, invE, zagg)

    cs = _final_stage([('agg', agg2), ('full', rp2), ('full', h)])
    pooled = (cs[0] / N_NODES) @ params['cr_W'] + params['cr_b']
    z = jax.nn.gelu(pooled @ params['p1_W'] + params['p1_b'], approximate=False)
    return z @ params['p2_W'] + params['p2_b']


# speedup vs baseline: 1.5684x; 1.0818x over previous
import functools

import jax
import jax.numpy as jnp
from jax import lax
from jax.experimental import pallas as pl
from jax.experimental.pallas import tpu as pltpu
from jax.experimental.pallas import tpu_sc as plsc

N_NODES = 50000
N_EDGES = 800000
D = 64
R = 8
DH = D // 2

NC = 2
NS = 16
L = 16

EB = 128
EPT = 392 * EB
EPAD = NS * EPT
NB = EPT // EB

AGG_ROWS = 50176
PER_S = AGG_ROWS // NS
CNT_ROWS = 401408
CNT_PER_S = CNT_ROWS // NS
EPW = EPAD // (NC * NS)
NBW = EPW // EB
ROWC = 8
NBO1 = NB // ROWC
ROWC2 = 4
NBO2 = NBW // ROWC2

_ROW_TILE = 1000

_mesh = plsc.VectorSubcoreMesh(core_axis_name="c", subcore_axis_name="s")

_BCAST_DNUMS = lax.GatherDimensionNumbers(
    offset_dims=(), collapsed_slice_dims=(0,), start_index_map=(0,))


def _lane_bcast(v, j):
    idx = jnp.full((L, 1), j, jnp.int32)
    return lax.gather(v, idx, _BCAST_DNUMS, (1,),
                      mode=lax.GatherScatterMode.PROMISE_IN_BOUNDS)


def _sc_count(comb, val, zblk):
    @functools.partial(
        pl.kernel,
        out_type=jax.ShapeDtypeStruct((CNT_ROWS,), jnp.float32),
        mesh=_mesh,
        compiler_params=pltpu.CompilerParams(use_tc_tiling_on_sc=False),
        scratch_types=[
            pltpu.VMEM_SHARED((CNT_ROWS,), jnp.float32),
            pltpu.VMEM((2, ROWC, EB), jnp.int32),
            pltpu.VMEM((2, ROWC, EB), jnp.float32),
            pltpu.SemaphoreType.DMA((2,)),
            pltpu.SemaphoreType.DMA((2,)),
        ],
    )
    def k(comb_h, val_h, z_h, out_h, cnt_sh, comb_v, val_v, insem, ssem):
        c = lax.axis_index("c")
        s = lax.axis_index("s")

        @pl.when(c == 0)
        def _():
            pltpu.sync_copy(z_h, cnt_sh.at[pl.ds(s * CNT_PER_S, CNT_PER_S)])
            plsc.subcore_barrier()

            def issue_in(ob, p):
                r0 = s * NB + ob * ROWC
                pltpu.async_copy(comb_h.at[pl.ds(r0, ROWC)], comb_v.at[p],
                                 insem.at[p])
                pltpu.async_copy(val_h.at[pl.ds(r0, ROWC)], val_v.at[p],
                                 insem.at[p])

            def wait_in(ob, p):
                r0 = s * NB + ob * ROWC
                pltpu.make_async_copy(comb_h.at[pl.ds(r0, ROWC)],
                                      comb_v.at[p], insem.at[p]).wait()
                pltpu.make_async_copy(val_h.at[pl.ds(r0, ROWC)],
                                      val_v.at[p], insem.at[p]).wait()

            def issue_sc(p):
                for i in range(ROWC):
                    pltpu.async_copy(val_v.at[p, i],
                                     cnt_sh.at[comb_v.at[p, i]],
                                     ssem.at[p], add=True)

            def wait_sc(p):
                for i in range(ROWC):
                    pltpu.make_async_copy(val_v.at[p, i],
                                          cnt_sh.at[comb_v.at[p, i]],
                                          ssem.at[p]).wait()

            issue_in(0, 0)

            def body(ob, _):
                for p in range(2):
                    b = 2 * ob + p
                    q = (p + 1) % 2
                    wait_in(b, p)

                    @pl.when(b >= 1)
                    def _():
                        wait_sc(q)

                    @pl.when(b < NBO1 - 1)
                    def _():
                        issue_in(b + 1, q)

                    issue_sc(p)
                return ()

            lax.fori_loop(0, NBO1 // 2, body, ())
            b = NBO1 - 1
            wait_in(b, b % 2)
            wait_sc((b + 1) % 2)
            issue_sc(b % 2)
            wait_sc(b % 2)
            plsc.subcore_barrier()
            pltpu.sync_copy(
                cnt_sh.at[pl.ds(s * CNT_PER_S, CNT_PER_S)],
                out_h.at[pl.ds(s * CNT_PER_S, CNT_PER_S)],
            )

    return k(comb.reshape(-1, EB), val.reshape(-1, EB), zblk)


def _sc_inv_gather(comb, invp):
    @functools.partial(
        pl.kernel,
        out_type=jax.ShapeDtypeStruct((EPAD // EB, EB), jnp.float32),
        mesh=_mesh,
        compiler_params=pltpu.CompilerParams(use_tc_tiling_on_sc=False),
        scratch_types=[
            pltpu.VMEM((2, ROWC2, EB), jnp.int32),
            pltpu.VMEM((2, ROWC2, EB), jnp.float32),
            pltpu.SemaphoreType.DMA((2,)),
            pltpu.SemaphoreType.DMA((2,)),
            pltpu.SemaphoreType.DMA((2,)),
        ],
    )
    def k(comb_h, invp_h, out_h, comb_v, inv_v, insem, gsem, osem):
        c = lax.axis_index("c")
        s = lax.axis_index("s")
        base = (s * NC + c) * NBW

        def issue_in(b, p):
            r0 = base + b * ROWC2
            pltpu.async_copy(comb_h.at[pl.ds(r0, ROWC2)], comb_v.at[p],
                             insem.at[p])

        def wait_in(b, p):
            r0 = base + b * ROWC2
            pltpu.make_async_copy(comb_h.at[pl.ds(r0, ROWC2)], comb_v.at[p],
                                  insem.at[p]).wait()

        def issue_g(p):
            for i in range(ROWC2):
                pltpu.async_copy(invp_h.at[comb_v.at[p, i]], inv_v.at[p, i],
                                 gsem.at[p])

        def wait_g(p):
            for i in range(ROWC2):
                pltpu.make_async_copy(invp_h.at[comb_v.at[p, i]],
                                      inv_v.at[p, i], gsem.at[p]).wait()

        def issue_out(b, p):
            r0 = base + b * ROWC2
            pltpu.async_copy(inv_v.at[p], out_h.at[pl.ds(r0, ROWC2)],
                             osem.at[p])

        def wait_out(b, p):
            r0 = base + b * ROWC2
            pltpu.make_async_copy(inv_v.at[p], out_h.at[pl.ds(r0, ROWC2)],
                                  osem.at[p]).wait()

        issue_in(0, 0)

        def body(g, _):
            for p in range(2):
                b = 2 * g + p
                q = (p + 1) % 2
                wait_in(b, p)

                @pl.when(b >= 1)
                def _():
                    wait_g(q)
                    issue_out(b - 1, q)

                @pl.when(b < NBO2 - 1)
                def _():
                    issue_in(b + 1, q)

                @pl.when(b >= 2)
                def _():
                    wait_out(b - 2, p)

                issue_g(p)
            return ()

        lax.fori_loop(0, NBO2 // 2, body, ())
        b = NBO2 - 1
        wait_in(b, 0)
        wait_g(1)
        issue_out(b - 1, 1)
        wait_out(b - 2, 0)
        issue_g(0)
        wait_g(0)
        issue_out(b, 0)
        wait_out(b - 1, 1)
        wait_out(b, 0)

    return k(comb.reshape(-1, EB), invp).reshape(EPAD)


def _sc_conv(tbl, epk, invE, zblk):
    @functools.partial(
        pl.kernel,
        out_type=jax.ShapeDtypeStruct((NC, AGG_ROWS, DH), jnp.float32),
        mesh=_mesh,
        compiler_params=pltpu.CompilerParams(use_tc_tiling_on_sc=False),
        scratch_types=[
            pltpu.VMEM_SHARED((AGG_ROWS, DH), jnp.float32),
            pltpu.VMEM((4, 2 * EB), jnp.int32),
            pltpu.VMEM((4, EB), jnp.float32),
            pltpu.VMEM((4, EB), jnp.int32),
            pltpu.VMEM((4, EB), jnp.int32),
            pltpu.VMEM((2, EB, DH), jnp.float32),
            pltpu.VMEM((2, EB, DH), jnp.float32),
            pltpu.SemaphoreType.DMA((4,)),
            pltpu.SemaphoreType.DMA((4,)),
            pltpu.SemaphoreType.DMA((4,)),
        ],
    )
    def k(tbl_h, epk_h, inv_h, z_h, out_h,
          agg_sh, in_v, inf_v, idx_v, sidx_v, rows_v, srow_v,
          insem, gsem, ssem):
        c = lax.axis_index("c")
        s = lax.axis_index("s")

        pltpu.sync_copy(z_h, agg_sh.at[pl.ds(s * PER_S, PER_S)])
        plsc.subcore_barrier()

        def issue_in(b, j):
            e0 = (s * NB + b) * (2 * EB)
            pltpu.async_copy(epk_h.at[pl.ds(e0, 2 * EB)], in_v.at[j],
                             insem.at[j])
            f0 = (s * NB + b) * EB
            pltpu.async_copy(inv_h.at[pl.ds(f0, EB)], inf_v.at[j],
                             insem.at[j])

        def wait_in(b, j):
            e0 = (s * NB + b) * (2 * EB)
            pltpu.make_async_copy(epk_h.at[pl.ds(e0, 2 * EB)], in_v.at[j],
                                  insem.at[j]).wait()
            f0 = (s * NB + b) * EB
            pltpu.make_async_copy(inv_h.at[pl.ds(f0, EB)], inf_v.at[j],
                                  insem.at[j]).wait()

        def prep(j):
            for kk in range(EB // L):
                sl = pl.ds(kk * L, L)
                idx_v[j, sl] = in_v[j, sl] + c
                sidx_v[j, sl] = in_v[j, pl.ds(EB + kk * L, L)]

        def issue_gather(j):
            pltpu.async_copy(tbl_h.at[idx_v.at[j]], rows_v.at[j % 2],
                             gsem.at[j])

        def wait_gather(j):
            pltpu.make_async_copy(tbl_h.at[idx_v.at[j]], rows_v.at[j % 2],
                                  gsem.at[j]).wait()

        def scale(j):
            j2 = j % 2
            for kk in range(EB // L):
                iv = inf_v[j, pl.ds(kk * L, L)]
                for jj in range(L):
                    e = kk * L + jj
                    bc = _lane_bcast(iv, jj)
                    srow_v[j2, e, pl.ds(0, L)] = \
                        rows_v[j2, e, pl.ds(0, L)] * bc
                    srow_v[j2, e, pl.ds(L, L)] = \
                        rows_v[j2, e, pl.ds(L, L)] * bc

        def issue_scatter(j):
            pltpu.async_copy(srow_v.at[j % 2], agg_sh.at[sidx_v.at[j]],
                             ssem.at[j], add=True)

        def wait_scatter(j):
            pltpu.make_async_copy(srow_v.at[j % 2], agg_sh.at[sidx_v.at[j]],
                                  ssem.at[j]).wait()

        issue_in(0, 0)
        issue_in(1, 1)
        issue_in(2, 2)
        wait_in(0, 0)
        prep(0)
        issue_gather(0)

        def body(g, _):
            for j in range(4):
                b = 4 * g + j
                jn = (j + 1) % 4
                wait_in(b + 1, jn)
                prep(jn)
                issue_gather(jn)
                wait_gather(j)

                @pl.when(b >= 2)
                def _():
                    wait_scatter((j + 2) % 4)

                scale(j)
                issue_scatter(j)

                @pl.when(b < NB - 3)
                def _():
                    issue_in(b + 3, (j + 3) % 4)
            return ()

        lax.fori_loop(0, NB // 4 - 1, body, ())
        issue_in(NB - 1, (NB - 1) % 4)
        for j in range(4):
            if j < 3:
                wait_in(NB - 3 + j, (j + 1) % 4)
                prep((j + 1) % 4)
                issue_gather((j + 1) % 4)
            wait_gather(j)
            wait_scatter((j + 2) % 4)
            scale(j)
            issue_scatter(j)
        wait_scatter(2)
        wait_scatter(3)

        plsc.subcore_barrier()
        pltpu.sync_copy(
            agg_sh.at[pl.ds(s * PER_S, PER_S)],
            out_h.at[c, pl.ds(s * PER_S, PER_S)],
        )

    return k(tbl, epk, invE, zblk)


def _sum_parts(parts, refs):
    h = None
    for (tag, _), r in zip(parts, refs):
        t = (jnp.concatenate([r[0], r[1]], axis=-1) if tag == 'agg'
             else r[...])
        h = t if h is None else h + t
    return h


def _part_spec(tag):
    if tag == 'agg':
        return pl.BlockSpec((NC, _ROW_TILE, DH), lambda i: (0, i, 0))
    return pl.BlockSpec((_ROW_TILE, D), lambda i: (i, 0))


def _dense_stage(parts, lng, lnb, Wst, root, bias, want_hmat):
    n_parts = len(parts)

    def body(*refs):
        pr = refs[:n_parts]
        lng_r, lnb_r, w_r, root_r, bias_r = refs[n_parts:n_parts + 5]
        o_t = refs[n_parts + 5]
        o_rp = refs[n_parts + 6]
        h = _sum_parts(parts, pr)
        if want_hmat:
            refs[n_parts + 7][...] = h
        m = jnp.mean(h, axis=-1, keepdims=True)
        v = jnp.mean((h - m) * (h - m), axis=-1, keepdims=True)
        xn = (h - m) * jax.lax.rsqrt(v + 1e-5) * lng_r[...] + lnb_r[...]
        y = xn * 0.5 * (1.0 + jax.lax.erf(xn / jnp.sqrt(2.0).astype(xn.dtype)))
        for ri in range(R):
            o_t[ri] = jnp.dot(y, w_r[ri], preferred_element_type=jnp.float32)
        o_rp[...] = jnp.dot(y, root_r[...],
                            preferred_element_type=jnp.float32) + bias_r[...]

    in_specs = [_part_spec(tag) for tag, _ in parts] + [
        pl.BlockSpec((1, D), lambda i: (0, 0)),
        pl.BlockSpec((1, D), lambda i: (0, 0)),
        pl.BlockSpec((R, D, D), lambda i: (0, 0, 0)),
        pl.BlockSpec((D, D), lambda i: (0, 0)),
        pl.BlockSpec((1, D), lambda i: (0, 0)),
    ]
    out_shape = [jax.ShapeDtypeStruct((R, N_NODES, D), jnp.float32),
                 jax.ShapeDtypeStruct((N_NODES, D), jnp.float32)]
    out_specs = [pl.BlockSpec((R, _ROW_TILE, D), lambda i: (0, i, 0)),
                 pl.BlockSpec((_ROW_TILE, D), lambda i: (i, 0))]
    if want_hmat:
        out_shape.append(jax.ShapeDtypeStruct((N_NODES, D), jnp.float32))
        out_specs.append(pl.BlockSpec((_ROW_TILE, D), lambda i: (i, 0)))
    return pl.pallas_call(
        body,
        grid=(N_NODES // _ROW_TILE,),
        in_specs=in_specs,
        out_specs=out_specs,
        out_shape=out_shape,
    )(*[a for _, a in parts], lng.reshape(1, D), lnb.reshape(1, D),
      Wst, root, bias.reshape(1, D))


def _final_stage(parts):
    n_parts = len(parts)

    def body(*refs):
        o_cs = refs[n_parts]
        h = _sum_parts(parts, refs[:n_parts])

        @pl.when(pl.program_id(0) == 0)
        def _():
            o_cs[...] = jnp.zeros((1, D), jnp.float32)

        o_cs[...] += jnp.sum(h, axis=0, keepdims=True)

    return pl.pallas_call(
        body,
        grid=(N_NODES // _ROW_TILE,),
        in_specs=[_part_spec(tag) for tag, _ in parts],
        out_specs=pl.BlockSpec((1, D), lambda i: (0, 0)),
        out_shape=jax.ShapeDtypeStruct((1, D), jnp.float32),
    )(*[a for _, a in parts])


def kernel(x, edge_index, edge_attr, params):
    src = edge_index[0].astype(jnp.int32)
    dst = edge_index[1].astype(jnp.int32)
    et = edge_attr.astype(jnp.int32)

    pad = EPAD - N_EDGES
    comb = jnp.concatenate([dst * R + et, jnp.full((pad,), N_NODES * R, jnp.int32)])
    g2 = jnp.concatenate([(et * N_NODES + src) * NC, jnp.zeros((pad,), jnp.int32)])
    dstp = jnp.concatenate([dst, jnp.full((pad,), N_NODES, jnp.int32)])
    val = jnp.concatenate([jnp.ones((N_EDGES,), jnp.float32),
                           jnp.zeros((pad,), jnp.float32)])

    zcnt = jnp.zeros((CNT_PER_S,), jnp.float32)
    zagg = jnp.zeros((PER_S, DH), jnp.float32)

    cnt = _sc_count(comb, val, zcnt)
    invp = 1.0 / jnp.maximum(cnt, 1.0)
    invE = _sc_inv_gather(comb, invp)
    epk = jnp.stack([g2.reshape(-1, EB), dstp.reshape(-1, EB)],
                    axis=1).reshape(-1)

    emb = params['emb']
    h = jnp.where((x[:, None] == 1), emb[1][None, :], emb[0][None, :])

    for bi, p in enumerate(params['blocks']):
        if bi == 0:
            parts1 = [('full', h)]
        else:
            parts1 = [('agg', agg2), ('full', rp2), ('full', h)]
        o1 = _dense_stage(parts1, p['ln1_g'], p['ln1_b'], p['W1'],
                          p['root1'], p['b1'], want_hmat=(bi > 0))
        if bi > 0:
            t1, rp1, h = o1
        else:
            t1, rp1 = o1
        agg1 = _sc_conv(t1.reshape(R * N_NODES * NC, DH), epk, invE, zagg)
        t2, rp2 = _dense_stage([('agg', agg1), ('full', rp1)],
                               p['ln2_g'], p['ln2_b'], p['W2'],
                               p['root2'], p['b2'], want_hmat=False)
        agg2 = _sc_conv(t2.reshape(R * N_NODES * NC, DH), epk, invE, zagg)

    cs = _final_stage([('agg', agg2), ('full', rp2), ('full', h)])
    pooled = (cs[0] / N_NODES) @ params['cr_W'] + params['cr_b']
    z = jax.nn.gelu(pooled @ params['p1_W'] + params['p1_b'], approximate=False)
    return z @ params['p2_W'] + params['p2_b']


# speedup vs baseline: 1.6219x; 1.0341x over previous
import functools

import jax
import jax.numpy as jnp
from jax import lax
from jax.experimental import pallas as pl
from jax.experimental.pallas import tpu as pltpu
from jax.experimental.pallas import tpu_sc as plsc

N_NODES = 50000
N_EDGES = 800000
D = 64
R = 8
DH = D // 2

NC = 2
NS = 16
L = 16

EB = 128
EPT = 392 * EB
EPAD = NS * EPT
NB = EPT // EB

AGG_ROWS = 50176
PER_S = AGG_ROWS // NS
CNT_ROWS = 401408
CNT_PER_S = CNT_ROWS // NS
EPW = EPAD // (NC * NS)
NBW = EPW // EB
ROWC = 8
NBO1 = NB // ROWC
ROWC2 = 4
NBO2 = NBW // ROWC2

_ROW_TILE = 2000

_mesh = plsc.VectorSubcoreMesh(core_axis_name="c", subcore_axis_name="s")

_BCAST_DNUMS = lax.GatherDimensionNumbers(
    offset_dims=(), collapsed_slice_dims=(0,), start_index_map=(0,))


def _lane_bcast(v, j):
    idx = jnp.full((L, 1), j, jnp.int32)
    return lax.gather(v, idx, _BCAST_DNUMS, (1,),
                      mode=lax.GatherScatterMode.PROMISE_IN_BOUNDS)


def _sc_count(comb, val, zblk):
    @functools.partial(
        pl.kernel,
        out_type=jax.ShapeDtypeStruct((CNT_ROWS,), jnp.float32),
        mesh=_mesh,
        compiler_params=pltpu.CompilerParams(use_tc_tiling_on_sc=False),
        scratch_types=[
            pltpu.VMEM_SHARED((CNT_ROWS,), jnp.float32),
            pltpu.VMEM((2, ROWC, EB), jnp.int32),
            pltpu.VMEM((2, ROWC, EB), jnp.float32),
            pltpu.SemaphoreType.DMA((2,)),
            pltpu.SemaphoreType.DMA((2,)),
        ],
    )
    def k(comb_h, val_h, z_h, out_h, cnt_sh, comb_v, val_v, insem, ssem):
        c = lax.axis_index("c")
        s = lax.axis_index("s")

        @pl.when(c == 0)
        def _():
            pltpu.sync_copy(z_h, cnt_sh.at[pl.ds(s * CNT_PER_S, CNT_PER_S)])
            plsc.subcore_barrier()

            def issue_in(ob, p):
                r0 = s * NB + ob * ROWC
                pltpu.async_copy(comb_h.at[pl.ds(r0, ROWC)], comb_v.at[p],
                                 insem.at[p])
                pltpu.async_copy(val_h.at[pl.ds(r0, ROWC)], val_v.at[p],
                                 insem.at[p])

            def wait_in(ob, p):
                r0 = s * NB + ob * ROWC
                pltpu.make_async_copy(comb_h.at[pl.ds(r0, ROWC)],
                                      comb_v.at[p], insem.at[p]).wait()
                pltpu.make_async_copy(val_h.at[pl.ds(r0, ROWC)],
                                      val_v.at[p], insem.at[p]).wait()

            def issue_sc(p):
                for i in range(ROWC):
                    pltpu.async_copy(val_v.at[p, i],
                                     cnt_sh.at[comb_v.at[p, i]],
                                     ssem.at[p], add=True)

            def wait_sc(p):
                for i in range(ROWC):
                    pltpu.make_async_copy(val_v.at[p, i],
                                          cnt_sh.at[comb_v.at[p, i]],
                                          ssem.at[p]).wait()

            issue_in(0, 0)

            def body(ob, _):
                for p in range(2):
                    b = 2 * ob + p
                    q = (p + 1) % 2
                    wait_in(b, p)

                    @pl.when(b >= 1)
                    def _():
                        wait_sc(q)

                    @pl.when(b < NBO1 - 1)
                    def _():
                        issue_in(b + 1, q)

                    issue_sc(p)
                return ()

            lax.fori_loop(0, NBO1 // 2, body, ())
            b = NBO1 - 1
            wait_in(b, b % 2)
            wait_sc((b + 1) % 2)
            issue_sc(b % 2)
            wait_sc(b % 2)
            plsc.subcore_barrier()
            pltpu.sync_copy(
                cnt_sh.at[pl.ds(s * CNT_PER_S, CNT_PER_S)],
                out_h.at[pl.ds(s * CNT_PER_S, CNT_PER_S)],
            )

    return k(comb.reshape(-1, EB), val.reshape(-1, EB), zblk)


def _sc_inv_gather(comb, invp):
    @functools.partial(
        pl.kernel,
        out_type=jax.ShapeDtypeStruct((EPAD // EB, EB), jnp.float32),
        mesh=_mesh,
        compiler_params=pltpu.CompilerParams(use_tc_tiling_on_sc=False),
        scratch_types=[
            pltpu.VMEM((2, ROWC2, EB), jnp.int32),
            pltpu.VMEM((2, ROWC2, EB), jnp.float32),
            pltpu.SemaphoreType.DMA((2,)),
            pltpu.SemaphoreType.DMA((2,)),
            pltpu.SemaphoreType.DMA((2,)),
        ],
    )
    def k(comb_h, invp_h, out_h, comb_v, inv_v, insem, gsem, osem):
        c = lax.axis_index("c")
        s = lax.axis_index("s")
        base = (s * NC + c) * NBW

        def issue_in(b, p):
            r0 = base + b * ROWC2
            pltpu.async_copy(comb_h.at[pl.ds(r0, ROWC2)], comb_v.at[p],
                             insem.at[p])

        def wait_in(b, p):
            r0 = base + b * ROWC2
            pltpu.make_async_copy(comb_h.at[pl.ds(r0, ROWC2)], comb_v.at[p],
                                  insem.at[p]).wait()

        def issue_g(p):
            for i in range(ROWC2):
                pltpu.async_copy(invp_h.at[comb_v.at[p, i]], inv_v.at[p, i],
                                 gsem.at[p])

        def wait_g(p):
            for i in range(ROWC2):
                pltpu.make_async_copy(invp_h.at[comb_v.at[p, i]],
                                      inv_v.at[p, i], gsem.at[p]).wait()

        def issue_out(b, p):
            r0 = base + b * ROWC2
            pltpu.async_copy(inv_v.at[p], out_h.at[pl.ds(r0, ROWC2)],
                             osem.at[p])

        def wait_out(b, p):
            r0 = base + b * ROWC2
            pltpu.make_async_copy(inv_v.at[p], out_h.at[pl.ds(r0, ROWC2)],
                                  osem.at[p]).wait()

        issue_in(0, 0)

        def body(g, _):
            for p in range(2):
                b = 2 * g + p
                q = (p + 1) % 2
                wait_in(b, p)

                @pl.when(b >= 1)
                def _():
                    wait_g(q)
                    issue_out(b - 1, q)

                @pl.when(b < NBO2 - 1)
                def _():
                    issue_in(b + 1, q)

                @pl.when(b >= 2)
                def _():
                    wait_out(b - 2, p)

                issue_g(p)
            return ()

        lax.fori_loop(0, NBO2 // 2, body, ())
        b = NBO2 - 1
        wait_in(b, 0)
        wait_g(1)
        issue_out(b - 1, 1)
        wait_out(b - 2, 0)
        issue_g(0)
        wait_g(0)
        issue_out(b, 0)
        wait_out(b - 1, 1)
        wait_out(b, 0)

    return k(comb.reshape(-1, EB), invp).reshape(EPAD)


def _sc_conv(tbl, epk, invE, zblk):
    @functools.partial(
        pl.kernel,
        out_type=jax.ShapeDtypeStruct((NC, AGG_ROWS, DH), jnp.float32),
        mesh=_mesh,
        compiler_params=pltpu.CompilerParams(use_tc_tiling_on_sc=False),
        scratch_types=[
            pltpu.VMEM_SHARED((AGG_ROWS, DH), jnp.float32),
            pltpu.VMEM((4, 2 * EB), jnp.int32),
            pltpu.VMEM((4, EB), jnp.float32),
            pltpu.VMEM((4, EB), jnp.int32),
            pltpu.VMEM((4, EB), jnp.int32),
            pltpu.VMEM((2, EB, DH), jnp.float32),
            pltpu.VMEM((2, EB, DH), jnp.float32),
            pltpu.SemaphoreType.DMA((4,)),
            pltpu.SemaphoreType.DMA((4,)),
            pltpu.SemaphoreType.DMA((4,)),
        ],
    )
    def k(tbl_h, epk_h, inv_h, z_h, out_h,
          agg_sh, in_v, inf_v, idx_v, sidx_v, rows_v, srow_v,
          insem, gsem, ssem):
        c = lax.axis_index("c")
        s = lax.axis_index("s")

        pltpu.sync_copy(z_h, agg_sh.at[pl.ds(s * PER_S, PER_S)])
        plsc.subcore_barrier()

        def issue_in(b, j):
            e0 = (s * NB + b) * (2 * EB)
            pltpu.async_copy(epk_h.at[pl.ds(e0, 2 * EB)], in_v.at[j],
                             insem.at[j])
            f0 = (s * NB + b) * EB
            pltpu.async_copy(inv_h.at[pl.ds(f0, EB)], inf_v.at[j],
                             insem.at[j])

        def wait_in(b, j):
            e0 = (s * NB + b) * (2 * EB)
            pltpu.make_async_copy(epk_h.at[pl.ds(e0, 2 * EB)], in_v.at[j],
                                  insem.at[j]).wait()
            f0 = (s * NB + b) * EB
            pltpu.make_async_copy(inv_h.at[pl.ds(f0, EB)], inf_v.at[j],
                                  insem.at[j]).wait()

        def prep(j):
            for kk in range(EB // L):
                sl = pl.ds(kk * L, L)
                idx_v[j, sl] = in_v[j, sl] + c
                sidx_v[j, sl] = in_v[j, pl.ds(EB + kk * L, L)]

        def issue_gather(j):
            pltpu.async_copy(tbl_h.at[idx_v.at[j]], rows_v.at[j % 2],
                             gsem.at[j])

        def wait_gather(j):
            pltpu.make_async_copy(tbl_h.at[idx_v.at[j]], rows_v.at[j % 2],
                                  gsem.at[j]).wait()

        def scale(j):
            j2 = j % 2
            for kk in range(EB // L):
                iv = inf_v[j, pl.ds(kk * L, L)]
                for jj in range(L):
                    e = kk * L + jj
                    bc = _lane_bcast(iv, jj)
                    srow_v[j2, e, pl.ds(0, L)] = \
                        rows_v[j2, e, pl.ds(0, L)] * bc
                    srow_v[j2, e, pl.ds(L, L)] = \
                        rows_v[j2, e, pl.ds(L, L)] * bc

        def issue_scatter(j):
            pltpu.async_copy(srow_v.at[j % 2], agg_sh.at[sidx_v.at[j]],
                             ssem.at[j], add=True)

        def wait_scatter(j):
            pltpu.make_async_copy(srow_v.at[j % 2], agg_sh.at[sidx_v.at[j]],
                                  ssem.at[j]).wait()

        issue_in(0, 0)
        issue_in(1, 1)
        issue_in(2, 2)
        wait_in(0, 0)
        prep(0)
        issue_gather(0)

        def body(g, _):
            for j in range(4):
                b = 4 * g + j
                jn = (j + 1) % 4
                wait_in(b + 1, jn)
                prep(jn)
                issue_gather(jn)
                wait_gather(j)

                @pl.when(b >= 2)
                def _():
                    wait_scatter((j + 2) % 4)

                scale(j)
                issue_scatter(j)

                @pl.when(b < NB - 3)
                def _():
                    issue_in(b + 3, (j + 3) % 4)
            return ()

        lax.fori_loop(0, NB // 4 - 1, body, ())
        issue_in(NB - 1, (NB - 1) % 4)
        for j in range(4):
            if j < 3:
                wait_in(NB - 3 + j, (j + 1) % 4)
                prep((j + 1) % 4)
                issue_gather((j + 1) % 4)
            wait_gather(j)
            wait_scatter((j + 2) % 4)
            scale(j)
            issue_scatter(j)
        wait_scatter(2)
        wait_scatter(3)

        plsc.subcore_barrier()
        pltpu.sync_copy(
            agg_sh.at[pl.ds(s * PER_S, PER_S)],
            out_h.at[c, pl.ds(s * PER_S, PER_S)],
        )

    return k(tbl, epk, invE, zblk)


def _sum_parts(parts, refs):
    h = None
    for (tag, _), r in zip(parts, refs):
        t = (jnp.concatenate([r[0], r[1]], axis=-1) if tag == 'agg'
             else r[...])
        h = t if h is None else h + t
    return h


def _part_spec(tag):
    if tag == 'agg':
        return pl.BlockSpec((NC, _ROW_TILE, DH), lambda i: (0, i, 0))
    return pl.BlockSpec((_ROW_TILE, D), lambda i: (i, 0))


def _dense_stage(parts, lng, lnb, Wst, root, bias, want_hmat):
    n_parts = len(parts)

    def body(*refs):
        pr = refs[:n_parts]
        lng_r, lnb_r, w_r, root_r, bias_r = refs[n_parts:n_parts + 5]
        o_t = refs[n_parts + 5]
        o_rp = refs[n_parts + 6]
        h = _sum_parts(parts, pr)
        if want_hmat:
            refs[n_parts + 7][...] = h
        m = jnp.mean(h, axis=-1, keepdims=True)
        v = jnp.mean((h - m) * (h - m), axis=-1, keepdims=True)
        xn = (h - m) * jax.lax.rsqrt(v + 1e-5) * lng_r[...] + lnb_r[...]
        y = xn * 0.5 * (1.0 + jax.lax.erf(xn / jnp.sqrt(2.0).astype(xn.dtype)))
        for ri in range(R):
            o_t[ri] = jnp.dot(y, w_r[ri], preferred_element_type=jnp.float32)
        o_rp[...] = jnp.dot(y, root_r[...],
                            preferred_element_type=jnp.float32) + bias_r[...]

    in_specs = [_part_spec(tag) for tag, _ in parts] + [
        pl.BlockSpec((1, D), lambda i: (0, 0)),
        pl.BlockSpec((1, D), lambda i: (0, 0)),
        pl.BlockSpec((R, D, D), lambda i: (0, 0, 0)),
        pl.BlockSpec((D, D), lambda i: (0, 0)),
        pl.BlockSpec((1, D), lambda i: (0, 0)),
    ]
    out_shape = [jax.ShapeDtypeStruct((R, N_NODES, D), jnp.float32),
                 jax.ShapeDtypeStruct((N_NODES, D), jnp.float32)]
    out_specs = [pl.BlockSpec((R, _ROW_TILE, D), lambda i: (0, i, 0)),
                 pl.BlockSpec((_ROW_TILE, D), lambda i: (i, 0))]
    if want_hmat:
        out_shape.append(jax.ShapeDtypeStruct((N_NODES, D), jnp.float32))
        out_specs.append(pl.BlockSpec((_ROW_TILE, D), lambda i: (i, 0)))
    return pl.pallas_call(
        body,
        grid=(N_NODES // _ROW_TILE,),
        in_specs=in_specs,
        out_specs=out_specs,
        out_shape=out_shape,
    )(*[a for _, a in parts], lng.reshape(1, D), lnb.reshape(1, D),
      Wst, root, bias.reshape(1, D))


def _final_stage(parts):
    n_parts = len(parts)

    def body(*refs):
        o_cs = refs[n_parts]
        h = _sum_parts(parts, refs[:n_parts])

        @pl.when(pl.program_id(0) == 0)
        def _():
            o_cs[...] = jnp.zeros((1, D), jnp.float32)

        o_cs[...] += jnp.sum(h, axis=0, keepdims=True)

    return pl.pallas_call(
        body,
        grid=(N_NODES // _ROW_TILE,),
        in_specs=[_part_spec(tag) for tag, _ in parts],
        out_specs=pl.BlockSpec((1, D), lambda i: (0, 0)),
        out_shape=jax.ShapeDtypeStruct((1, D), jnp.float32),
    )(*[a for _, a in parts])


def kernel(x, edge_index, edge_attr, params):
    src = edge_index[0].astype(jnp.int32)
    dst = edge_index[1].astype(jnp.int32)
    et = edge_attr.astype(jnp.int32)

    pad = EPAD - N_EDGES
    comb = jnp.concatenate([dst * R + et, jnp.full((pad,), N_NODES * R, jnp.int32)])
    g2 = jnp.concatenate([(et * N_NODES + src) * NC, jnp.zeros((pad,), jnp.int32)])
    dstp = jnp.concatenate([dst, jnp.full((pad,), N_NODES, jnp.int32)])
    val = jnp.concatenate([jnp.ones((N_EDGES,), jnp.float32),
                           jnp.zeros((pad,), jnp.float32)])

    zcnt = jnp.zeros((CNT_PER_S,), jnp.float32)
    zagg = jnp.zeros((PER_S, DH), jnp.float32)

    cnt = _sc_count(comb, val, zcnt)
    invp = 1.0 / jnp.maximum(cnt, 1.0)
    invE = _sc_inv_gather(comb, invp)
    epk = jnp.stack([g2.reshape(-1, EB), dstp.reshape(-1, EB)],
                    axis=1).reshape(-1)

    emb = params['emb']
    h = jnp.where((x[:, None] == 1), emb[1][None, :], emb[0][None, :])

    for bi, p in enumerate(params['blocks']):
        if bi == 0:
            parts1 = [('full', h)]
        else:
            parts1 = [('agg', agg2), ('full', rp2), ('full', h)]
        o1 = _dense_stage(parts1, p['ln1_g'], p['ln1_b'], p['W1'],
                          p['root1'], p['b1'], want_hmat=(bi > 0))
        if bi > 0:
            t1, rp1, h = o1
        else:
            t1, rp1 = o1
        agg1 = _sc_conv(t1.reshape(R * N_NODES * NC, DH), epk, invE, zagg)
        t2, rp2 = _dense_stage([('agg', agg1), ('full', rp1)],
                               p['ln2_g'], p['ln2_b'], p['W2'],
                               p['root2'], p['b2'], want_hmat=False)
        agg2 = _sc_conv(t2.reshape(R * N_NODES * NC, DH), epk, invE, zagg)

    cs = _final_stage([('agg', agg2), ('full', rp2), ('full', h)])
    pooled = (cs[0] / N_NODES) @ params['cr_W'] + params['cr_b']
    z = jax.nn.gelu(pooled @ params['p1_W'] + params['p1_b'], approximate=False)
    return z @ params['p2_W'] + params['p2_b']
